# speedup vs baseline: 4.6981x; 4.6981x over previous
import dataclasses
import functools

import jax
import jax.numpy as jnp
from jax import lax
from jax.experimental import pallas as pl
from jax.experimental.pallas import tpu as pltpu
from jax.experimental.pallas import tpu_sc as plsc

_NC = 2
_NS = 16
_NW = _NC * _NS
_L = 16
_MAGIC = 12582912.0


@functools.partial(jax.jit, static_argnums=(2, 3))
def _sc_lookup(t, table_padded, n, scale):
    chunk = n // _NW
    mesh = plsc.VectorSubcoreMesh(core_axis_name="c", subcore_axis_name="s")
    cp = pltpu.CompilerParams()
    if "needs_layout_passes" in pltpu.CompilerParams.__dataclass_fields__:
        cp = dataclasses.replace(cp, needs_layout_passes=False)

    @functools.partial(
        pl.kernel,
        out_type=jax.ShapeDtypeStruct((n,), jnp.float32),
        mesh=mesh,
        compiler_params=cp,
        scratch_types=[
            pltpu.VMEM((table_padded.shape[0],), jnp.float32),
            pltpu.VMEM((chunk,), jnp.float32),
            pltpu.VMEM((chunk,), jnp.float32),
        ],
    )
    def k(t_hbm, tbl_hbm, out_hbm, tbl_v, t_v, o_v):
        wid = lax.axis_index("s") * _NC + lax.axis_index("c")
        base = wid * chunk
        pltpu.sync_copy(tbl_hbm, tbl_v)
        pltpu.sync_copy(t_hbm.at[pl.ds(base, chunk)], t_v)

        @pl.loop(0, chunk, step=_L)
        def _(i):
            x = t_v[pl.ds(i, _L)] * jnp.float32(scale)
            r = (x + jnp.float32(_MAGIC)) - jnp.float32(_MAGIC)
            idx = r.astype(jnp.int32)
            o_v[pl.ds(i, _L)] = plsc.load_gather(tbl_v, [idx])

        pltpu.sync_copy(o_v, out_hbm.at[pl.ds(base, chunk)])

    return k(t, table_padded)


def kernel(t, type, alpha, sigma):
    T = alpha.shape[0] - 1
    table = alpha if type == 'alpha' else sigma
    pad = (-table.shape[0]) % 64
    table_padded = jnp.pad(table, (0, pad))
    return _sc_lookup(t, table_padded, t.shape[0], float(T))


# speedup vs baseline: 4.7699x; 1.0153x over previous
import dataclasses
import functools

import jax
import jax.numpy as jnp
from jax import lax
from jax.experimental import pallas as pl
from jax.experimental.pallas import tpu as pltpu
from jax.experimental.pallas import tpu_sc as plsc

_NC = 2
_NS = 16
_NW = _NC * _NS
_L = 16
_MAGIC = 12582912.0


@functools.partial(jax.jit, static_argnums=(2, 3))
def _sc_lookup(t, table_padded, n, scale):
    chunk = n // _NW
    mesh = plsc.VectorSubcoreMesh(core_axis_name="c", subcore_axis_name="s")
    cp = pltpu.CompilerParams()
    if "needs_layout_passes" in pltpu.CompilerParams.__dataclass_fields__:
        cp = dataclasses.replace(cp, needs_layout_passes=False)

    @functools.partial(
        pl.kernel,
        out_type=jax.ShapeDtypeStruct((n,), jnp.float32),
        mesh=mesh,
        compiler_params=cp,
        scratch_types=[
            pltpu.VMEM((table_padded.shape[0],), jnp.float32),
            pltpu.VMEM((chunk,), jnp.float32),
            pltpu.VMEM((chunk,), jnp.float32),
            pltpu.SemaphoreType.DMA,
            pltpu.SemaphoreType.DMA,
        ],
    )
    def k(t_hbm, tbl_hbm, out_hbm, tbl_v, t_v, o_v, sem0, sem1):
        wid = lax.axis_index("s") * _NC + lax.axis_index("c")
        base = wid * chunk
        cp_tbl = pltpu.async_copy(tbl_hbm, tbl_v, sem0)
        cp_t = pltpu.async_copy(t_hbm.at[pl.ds(base, chunk)], t_v, sem1)
        cp_t.wait()
        cp_tbl.wait()

        @plsc.parallel_loop(0, chunk, step=_L, unroll=4)
        def _(i):
            x = t_v[pl.ds(i, _L)] * jnp.float32(scale)
            r = (x + jnp.float32(_MAGIC)) - jnp.float32(_MAGIC)
            idx = r.astype(jnp.int32)
            o_v[pl.ds(i, _L)] = plsc.load_gather(tbl_v, [idx])

        pltpu.sync_copy(o_v, out_hbm.at[pl.ds(base, chunk)])

    return k(t, table_padded)


def kernel(t, type, alpha, sigma):
    T = alpha.shape[0] - 1
    table = alpha if type == 'alpha' else sigma
    return _sc_lookup(t, table, t.shape[0], float(T))


# speedup vs baseline: 5.1264x; 1.0747x over previous
import dataclasses
import functools

import jax
import jax.numpy as jnp
from jax import lax
from jax.experimental import pallas as pl
from jax.experimental.pallas import tpu as pltpu
from jax.experimental.pallas import tpu_sc as plsc

_NC = 1
_NS = 16
_NW = _NC * _NS
_L = 16
_MAGIC = 12582912.0


@functools.partial(jax.jit, static_argnums=(2, 3))
def _sc_lookup(t, table_padded, n, scale):
    chunk = n // _NW
    mesh = plsc.VectorSubcoreMesh(
        core_axis_name="c", subcore_axis_name="s", num_cores=_NC)
    cp = pltpu.CompilerParams()
    if "needs_layout_passes" in pltpu.CompilerParams.__dataclass_fields__:
        cp = dataclasses.replace(cp, needs_layout_passes=False)

    @functools.partial(
        pl.kernel,
        out_type=jax.ShapeDtypeStruct((n,), jnp.float32),
        mesh=mesh,
        compiler_params=cp,
        scratch_types=[
            pltpu.VMEM((table_padded.shape[0],), jnp.float32),
            pltpu.VMEM((chunk,), jnp.float32),
            pltpu.VMEM((chunk,), jnp.float32),
            pltpu.SemaphoreType.DMA,
            pltpu.SemaphoreType.DMA,
        ],
    )
    def k(t_hbm, tbl_hbm, out_hbm, tbl_v, t_v, o_v, sem0, sem1):
        wid = lax.axis_index("s") * _NC + lax.axis_index("c")
        base = wid * chunk
        cp_tbl = pltpu.async_copy(tbl_hbm, tbl_v, sem0)
        cp_t = pltpu.async_copy(t_hbm.at[pl.ds(base, chunk)], t_v, sem1)
        cp_t.wait()
        cp_tbl.wait()

        @plsc.parallel_loop(0, chunk, step=_L, unroll=4)
        def _(i):
            x = t_v[pl.ds(i, _L)] * jnp.float32(scale)
            r = (x + jnp.float32(_MAGIC)) - jnp.float32(_MAGIC)
            idx = r.astype(jnp.int32)
            o_v[pl.ds(i, _L)] = plsc.load_gather(tbl_v, [idx])

        pltpu.sync_copy(o_v, out_hbm.at[pl.ds(base, chunk)])

    return k(t, table_padded)


def kernel(t, type, alpha, sigma):
    T = alpha.shape[0] - 1
    table = alpha if type == 'alpha' else sigma
    return _sc_lookup(t, table, t.shape[0], float(T))


# speedup vs baseline: 5.1404x; 1.0027x over previous
import dataclasses
import functools

import jax
import jax.numpy as jnp
from jax import lax
from jax.experimental import pallas as pl
from jax.experimental.pallas import tpu as pltpu
from jax.experimental.pallas import tpu_sc as plsc

_NC = 1
_NS = 16
_NW = _NC * _NS
_L = 16
_MAGIC = 12582912.0


@functools.partial(jax.jit, static_argnums=(2, 3))
def _sc_lookup(t, table_padded, n, scale):
    chunk = n // _NW
    mesh = plsc.VectorSubcoreMesh(
        core_axis_name="c", subcore_axis_name="s", num_cores=_NC)
    cp = pltpu.CompilerParams()
    if "needs_layout_passes" in pltpu.CompilerParams.__dataclass_fields__:
        cp = dataclasses.replace(cp, needs_layout_passes=False)

    @functools.partial(
        pl.kernel,
        out_type=jax.ShapeDtypeStruct((n,), jnp.float32),
        mesh=mesh,
        compiler_params=cp,
        scratch_types=[
            pltpu.VMEM((table_padded.shape[0],), jnp.float32),
            pltpu.VMEM((chunk,), jnp.float32),
            pltpu.SemaphoreType.DMA,
            pltpu.SemaphoreType.DMA,
        ],
    )
    def k(t_hbm, tbl_hbm, out_hbm, tbl_v, t_v, sem0, sem1):
        wid = lax.axis_index("s") * _NC + lax.axis_index("c")
        base = wid * chunk
        cp_tbl = pltpu.async_copy(tbl_hbm, tbl_v, sem0)
        cp_t = pltpu.async_copy(t_hbm.at[pl.ds(base, chunk)], t_v, sem1)
        cp_t.wait()
        cp_tbl.wait()

        @plsc.parallel_loop(0, chunk, step=_L, unroll=8)
        def _(i):
            x = t_v[pl.ds(i, _L)] * jnp.float32(scale)
            r = (x + jnp.float32(_MAGIC)) - jnp.float32(_MAGIC)
            idx = r.astype(jnp.int32)
            t_v[pl.ds(i, _L)] = plsc.load_gather(tbl_v, [idx])

        pltpu.sync_copy(t_v, out_hbm.at[pl.ds(base, chunk)])

    return k(t, table_padded)


def kernel(t, type, alpha, sigma):
    T = alpha.shape[0] - 1
    table = alpha if type == 'alpha' else sigma
    return _sc_lookup(t, table, t.shape[0], float(T))
